# Initial kernel scaffold; baseline (speedup 1.0000x reference)
import jax
import jax.numpy as jnp
from jax.experimental import pallas as pl


def kernel(features, motifs, adj_feat, adj_motif, W_em, b_em, a_em, W_ef, b_ef, a_ef, Wm1, bm1, am1, Wm2, bm2, am2, Wm3, bm3, am3, Wf1, bf1, af1, Wf2, bf2, af2, Wf3, bf3, af3, Wt, bt, Wd, bd):
    raise NotImplementedError("write your pallas kernel here")


# speedup vs baseline: 1.4012x; 1.4012x over previous
import functools

import jax
import jax.numpy as jnp
from jax.experimental import pallas as pl

B = 16384
S = 5
FEAT = 256
MOT = 64
H = 64
T = 512
P = 8
E = T + P


def _prelu(x, a):
    return jnp.where(x >= 0, x, a * x)


def _fused_kernel(
    feat_ref, featp_ref, mot_ref, motp_ref, adjf_ref, adjfp_ref,
    adjm_ref, adjmp_ref,
    wem_ref, bem_ref, aem_ref, wef_ref, bef_ref, aef_ref,
    wm1_ref, bm1_ref, am1_ref, wm2_ref, bm2_ref, am2_ref,
    wm3_ref, bm3_ref, am3_ref,
    wf1_ref, bf1_ref, af1_ref, wf2_ref, bf2_ref, af2_ref,
    wf3_ref, bf3_ref, af3_ref,
    wt_ref, bt_ref, wd_ref, bd_ref,
    mrec_ref, frec_ref, sc0_ref, sc1_ref, sc2_ref,
):
    f32 = jnp.float32
    dot = functools.partial(jnp.dot, preferred_element_type=f32)

    adjf = jnp.concatenate([adjfp_ref[...], adjf_ref[...]], axis=0)
    adjm = jnp.concatenate([adjmp_ref[...], adjm_ref[...]], axis=0)

    wef = wef_ref[...]
    wem = wem_ref[...]
    bef = bef_ref[...]
    bem = bem_ref[...]
    aef = aef_ref[0, 0]
    aem = aem_ref[0, 0]

    fts_f = []
    fts_m = []
    for s in range(S):
        xf = jnp.concatenate(
            [featp_ref[:, s, :], feat_ref[:, s, :]], axis=0)
        xm = jnp.concatenate(
            [motp_ref[:, s, :], mot_ref[:, s, :]], axis=0)
        fts_f.append(dot(xf, wef))
        fts_m.append(dot(xm, wem))

    hf = []
    hm = []
    for s in range(S):
        accf = adjf[:, 5 * s + 0:5 * s + 1] * fts_f[0]
        accm = adjm[:, 5 * s + 0:5 * s + 1] * fts_m[0]
        for t in range(1, S):
            accf = accf + adjf[:, 5 * s + t:5 * s + t + 1] * fts_f[t]
            accm = accm + adjm[:, 5 * s + t:5 * s + t + 1] * fts_m[t]
        hf.append(_prelu(accf + bef, aef))
        hm.append(_prelu(accm + bem, aem))

    wt = wt_ref[...]
    bt = bt_ref[0, 0]
    readout = None
    for s in range(S - 1):
        d = hm[s] - hm[S - 1]
        sc = jax.nn.sigmoid(
            jnp.sum(d * wt, axis=1, keepdims=True) + bt)
        term = sc * hf[s]
        readout = term if readout is None else readout + term

    target = hf[S - 1][P:, :]
    u = dot(target, wd_ref[...])
    bd = bd_ref[0, 0]
    sc0_ref[...] = jnp.sum(u * readout[P:, :], axis=1, keepdims=True) + bd
    sc1_ref[...] = jnp.sum(
        u * readout[P - 1:P - 1 + T, :], axis=1, keepdims=True) + bd
    sc2_ref[...] = jnp.sum(
        u * readout[P - 2:P - 2 + T, :], axis=1, keepdims=True) + bd

    nm = jnp.concatenate([hm[0][P:], hm[1][P:], hm[2][P:]], axis=1)
    nf = jnp.concatenate([hf[0][P:], hf[1][P:], hf[2][P:]], axis=1)

    x = _prelu(dot(nm, wm1_ref[...]) + bm1_ref[...], am1_ref[0, 0])
    x = _prelu(dot(x, wm2_ref[...]) + bm2_ref[...], am2_ref[0, 0])
    mrec_ref[...] = _prelu(dot(x, wm3_ref[...]) + bm3_ref[...], am3_ref[0, 0])

    y = _prelu(dot(nf, wf1_ref[...]) + bf1_ref[...], af1_ref[0, 0])
    y = _prelu(dot(y, wf2_ref[...]) + bf2_ref[...], af2_ref[0, 0])
    frec_ref[...] = _prelu(dot(y, wf3_ref[...]) + bf3_ref[...], af3_ref[0, 0])


def kernel(features, motifs, adj_feat, adj_motif, W_em, b_em, a_em, W_ef,
           b_ef, a_ef, Wm1, bm1, am1, Wm2, bm2, am2, Wm3, bm3, am3, Wf1, bf1,
           af1, Wf2, bf2, af2, Wf3, bf3, af3, Wt, bt, Wd, bd):
    f32 = jnp.float32
    adjf2 = adj_feat.reshape(B, S * S)
    adjm2 = adj_motif.reshape(B, S * S)

    def row(v, n):
        return v.reshape(1, n).astype(f32)

    w_args = (
        W_em.T, row(b_em, H), row(a_em, 1),
        W_ef.T, row(b_ef, H), row(a_ef, 1),
        Wm1.T, row(bm1, H), row(am1, 1),
        Wm2.T, row(bm2, H), row(am2, 1),
        Wm3.T, row(bm3, MOT), row(am3, 1),
        Wf1.T, row(bf1, H), row(af1, 1),
        Wf2.T, row(bf2, H), row(af2, 1),
        Wf3.T, row(bf3, FEAT), row(af3, 1),
        Wt, row(bt, 1), Wd[0], row(bd, 1),
    )

    grid = (B // T,)
    nb_prev = B // P

    def main2(t):
        return (t, 0)

    def main3(t):
        return (t, 0, 0)

    def prev2(t):
        return ((t * (T // P) - 1) % nb_prev, 0)

    def prev3(t):
        return ((t * (T // P) - 1) % nb_prev, 0, 0)

    def const(shape):
        return pl.BlockSpec(shape, lambda t: (0,) * len(shape))

    in_specs = [
        pl.BlockSpec((T, S, FEAT), main3),
        pl.BlockSpec((P, S, FEAT), prev3),
        pl.BlockSpec((T, S, MOT), main3),
        pl.BlockSpec((P, S, MOT), prev3),
        pl.BlockSpec((T, S * S), main2),
        pl.BlockSpec((P, S * S), prev2),
        pl.BlockSpec((T, S * S), main2),
        pl.BlockSpec((P, S * S), prev2),
    ] + [const(w.shape) for w in w_args]

    out_specs = [
        pl.BlockSpec((T, MOT), main2),
        pl.BlockSpec((T, FEAT), main2),
        pl.BlockSpec((T, 1), main2),
        pl.BlockSpec((T, 1), main2),
        pl.BlockSpec((T, 1), main2),
    ]
    out_shape = [
        jax.ShapeDtypeStruct((B, MOT), f32),
        jax.ShapeDtypeStruct((B, FEAT), f32),
        jax.ShapeDtypeStruct((B, 1), f32),
        jax.ShapeDtypeStruct((B, 1), f32),
        jax.ShapeDtypeStruct((B, 1), f32),
    ]

    mrec, frec, sc0, sc1, sc2 = pl.pallas_call(
        _fused_kernel,
        grid=grid,
        in_specs=in_specs,
        out_specs=out_specs,
        out_shape=out_shape,
    )(features, features, motifs, motifs, adjf2, adjf2,
      adjm2, adjm2, *w_args)

    logits = jnp.concatenate([sc0, sc1, sc2], axis=0)
    return (logits, mrec, frec)


# speedup vs baseline: 1.5560x; 1.1104x over previous
import functools

import jax
import jax.numpy as jnp
from jax.experimental import pallas as pl

B = 16384
S = 5
FEAT = 256
MOT = 64
H = 64
T = 512
P = 8
E = T + P


def _fused_kernel(
    feat_ref, featp_ref, mot_ref, motp_ref, adjf_ref, adjfp_ref,
    adjm_ref, adjmp_ref,
    g25_ref, ones_r_ref, ones_c_ref,
    wef_ref, wem_ref, bef_ref, bem_ref, aef_ref, aem_ref,
    wt_ref, bt_ref, wd_ref, bd_ref,
    w1_ref, b1_ref, a1_ref, w2_ref, b2_ref, a2_ref, w3_ref, b3_ref, a3_ref,
    mrec_ref, frec_ref, sc0_ref, sc1_ref, sc2_ref,
):
    f32 = jnp.float32
    dot = functools.partial(jnp.dot, preferred_element_type=f32)

    adjf = jnp.concatenate([adjfp_ref[...], adjf_ref[...]], axis=0)
    adjm = jnp.concatenate([adjmp_ref[...], adjm_ref[...]], axis=0)

    xf = jnp.concatenate(
        [featp_ref[...], feat_ref[...]], axis=0)
    xm = jnp.concatenate(
        [motp_ref[...], mot_ref[...]], axis=0)
    xf_stack = jnp.concatenate(
        [xf[:, s * FEAT:(s + 1) * FEAT] for s in range(S)], axis=0)
    xm_stack = jnp.concatenate(
        [xm[:, s * MOT:(s + 1) * MOT] for s in range(S)], axis=0)
    fts_f = dot(xf_stack, wef_ref[...])
    fts_m = dot(xm_stack, wem_ref[...])

    bcf = dot(adjf, g25_ref[...])
    bcm = dot(adjm, g25_ref[...])

    bef = bef_ref[...]
    bem = bem_ref[...]
    aef = aef_ref[0, 0]
    aem = aem_ref[0, 0]
    hf = []
    hm = []
    for s in range(S):
        c0 = S * s
        accf = bcf[:, c0 * H:(c0 + 1) * H] * fts_f[:E]
        accm = bcm[:, c0 * H:(c0 + 1) * H] * fts_m[:E]
        for t in range(1, S):
            c = c0 + t
            accf += bcf[:, c * H:(c + 1) * H] * fts_f[t * E:(t + 1) * E]
            accm += bcm[:, c * H:(c + 1) * H] * fts_m[t * E:(t + 1) * E]
        accf += bef
        accm += bem
        hf.append(jnp.where(accf >= 0, accf, aef * accf))
        hm.append(jnp.where(accm >= 0, accm, aem * accm))

    hm4 = hm[S - 1]
    dstack = jnp.concatenate([hm[s] - hm4 for s in range(S - 1)], axis=0)
    z = dot(dstack, wt_ref[...]) + bt_ref[0, 0]
    sc = jax.nn.sigmoid(z)
    scb = dot(sc, ones_r_ref[...])
    readout = scb[:E] * hf[0]
    for s in range(1, S - 1):
        readout += scb[s * E:(s + 1) * E] * hf[s]

    u = dot(hf[S - 1][P:, :], wd_ref[...])
    rstack = jnp.concatenate(
        [readout[P:P + T], readout[P - 1:P - 1 + T], readout[P - 2:P - 2 + T]],
        axis=0)
    ustack = jnp.concatenate([u, u, u], axis=0)
    psum = dot(ustack * rstack, ones_c_ref[...]) + bd_ref[0, 0]
    sc0_ref[...] = psum[:T]
    sc1_ref[...] = psum[T:2 * T]
    sc2_ref[...] = psum[2 * T:]

    nmf = jnp.concatenate(
        [hm[0][P:], hm[1][P:], hm[2][P:],
         hf[0][P:], hf[1][P:], hf[2][P:]], axis=1)
    x = dot(nmf, w1_ref[...]) + b1_ref[...]
    x = jnp.where(x >= 0, x, a1_ref[...] * x)
    x = dot(x, w2_ref[...]) + b2_ref[...]
    x = jnp.where(x >= 0, x, a2_ref[...] * x)
    x = dot(x, w3_ref[...]) + b3_ref[...]
    x = jnp.where(x >= 0, x, a3_ref[...] * x)
    frec_ref[...] = x[:, :FEAT]
    mrec_ref[...] = x[:, FEAT:]


def kernel(features, motifs, adj_feat, adj_motif, W_em, b_em, a_em, W_ef,
           b_ef, a_ef, Wm1, bm1, am1, Wm2, bm2, am2, Wm3, bm3, am3, Wf1, bf1,
           af1, Wf2, bf2, af2, Wf3, bf3, af3, Wt, bt, Wd, bd):
    f32 = jnp.float32
    feat2 = features.reshape(B, S * FEAT)
    mot2 = motifs.reshape(B, S * MOT)
    adjf2 = adj_feat.reshape(B, S * S)
    adjm2 = adj_motif.reshape(B, S * S)

    def row(v, n):
        return v.reshape(1, n).astype(f32)

    g25 = jnp.kron(jnp.eye(S * S, dtype=f32), jnp.ones((1, H), f32))
    ones_r = jnp.ones((1, H), f32)
    ones_c = jnp.ones((H, 1), f32)

    zz = jnp.zeros((3 * H, H), f32)
    w1b = jnp.concatenate(
        [jnp.concatenate([Wm1.T, zz], axis=0),
         jnp.concatenate([zz, Wf1.T], axis=0)], axis=1)
    b1b = jnp.concatenate([row(bm1, H), row(bf1, H)], axis=1)
    a1b = jnp.concatenate(
        [jnp.broadcast_to(row(am1, 1), (1, H)),
         jnp.broadcast_to(row(af1, 1), (1, H))], axis=1)
    z2 = jnp.zeros((H, H), f32)
    w2b = jnp.concatenate(
        [jnp.concatenate([Wm2.T, z2], axis=0),
         jnp.concatenate([z2, Wf2.T], axis=0)], axis=1)
    b2b = jnp.concatenate([row(bm2, H), row(bf2, H)], axis=1)
    a2b = jnp.concatenate(
        [jnp.broadcast_to(row(am2, 1), (1, H)),
         jnp.broadcast_to(row(af2, 1), (1, H))], axis=1)
    w3b = jnp.concatenate(
        [jnp.concatenate([jnp.zeros((H, FEAT), f32), Wm3.T], axis=1),
         jnp.concatenate([Wf3.T, jnp.zeros((H, MOT), f32)], axis=1)],
        axis=0)
    b3b = jnp.concatenate([row(bf3, FEAT), row(bm3, MOT)], axis=1)
    a3b = jnp.concatenate(
        [jnp.broadcast_to(row(af3, 1), (1, FEAT)),
         jnp.broadcast_to(row(am3, 1), (1, MOT))], axis=1)

    w_args = (
        g25, ones_r, ones_c,
        W_ef.T, W_em.T, row(b_ef, H), row(b_em, H), row(a_ef, 1),
        row(a_em, 1),
        Wt.T, row(bt, 1), Wd[0], row(bd, 1),
        w1b, b1b, a1b, w2b, b2b, a2b, w3b, b3b, a3b,
    )

    grid = (B // T,)
    nb_prev = B // P

    def main2(t):
        return (t, 0)

    def prev2(t):
        return ((t * (T // P) - 1) % nb_prev, 0)

    def const(shape):
        return pl.BlockSpec(shape, lambda t: (0,) * len(shape))

    in_specs = [
        pl.BlockSpec((T, S * FEAT), main2),
        pl.BlockSpec((P, S * FEAT), prev2),
        pl.BlockSpec((T, S * MOT), main2),
        pl.BlockSpec((P, S * MOT), prev2),
        pl.BlockSpec((T, S * S), main2),
        pl.BlockSpec((P, S * S), prev2),
        pl.BlockSpec((T, S * S), main2),
        pl.BlockSpec((P, S * S), prev2),
    ] + [const(w.shape) for w in w_args]

    out_specs = [
        pl.BlockSpec((T, MOT), main2),
        pl.BlockSpec((T, FEAT), main2),
        pl.BlockSpec((T, 1), main2),
        pl.BlockSpec((T, 1), main2),
        pl.BlockSpec((T, 1), main2),
    ]
    out_shape = [
        jax.ShapeDtypeStruct((B, MOT), f32),
        jax.ShapeDtypeStruct((B, FEAT), f32),
        jax.ShapeDtypeStruct((B, 1), f32),
        jax.ShapeDtypeStruct((B, 1), f32),
        jax.ShapeDtypeStruct((B, 1), f32),
    ]

    mrec, frec, sc0, sc1, sc2 = pl.pallas_call(
        _fused_kernel,
        grid=grid,
        in_specs=in_specs,
        out_specs=out_specs,
        out_shape=out_shape,
    )(feat2, feat2, mot2, mot2, adjf2, adjf2, adjm2, adjm2, *w_args)

    logits = jnp.concatenate([sc0, sc1, sc2], axis=0)
    return (logits, mrec, frec)


# speedup vs baseline: 1.5689x; 1.0083x over previous
import functools

import jax
import jax.numpy as jnp
from jax.experimental import pallas as pl

B = 16384
S = 5
FEAT = 256
MOT = 64
H = 64
T = 1024
P = 8
E = T + P


def _fused_kernel(
    feat_ref, featp_ref, mot_ref, motp_ref, adjf_ref, adjfp_ref,
    adjm_ref, adjmp_ref,
    g25_ref, ones_r_ref, ones_c_ref,
    wef_ref, wem_ref, bef_ref, bem_ref, aef_ref, aem_ref,
    wt_ref, bt_ref, wd_ref, bd_ref,
    w1_ref, b1_ref, a1_ref, w2_ref, b2_ref, a2_ref, w3_ref, b3_ref, a3_ref,
    mrec_ref, frec_ref, sc0_ref, sc1_ref, sc2_ref,
):
    f32 = jnp.float32
    dot = functools.partial(jnp.dot, preferred_element_type=f32)

    adjf = jnp.concatenate([adjfp_ref[...], adjf_ref[...]], axis=0)
    adjm = jnp.concatenate([adjmp_ref[...], adjm_ref[...]], axis=0)

    xf = jnp.concatenate(
        [featp_ref[...], feat_ref[...]], axis=0)
    xm = jnp.concatenate(
        [motp_ref[...], mot_ref[...]], axis=0)
    xf_stack = jnp.concatenate(
        [xf[:, s * FEAT:(s + 1) * FEAT] for s in range(S)], axis=0)
    xm_stack = jnp.concatenate(
        [xm[:, s * MOT:(s + 1) * MOT] for s in range(S)], axis=0)
    fts_f = dot(xf_stack, wef_ref[...])
    fts_m = dot(xm_stack, wem_ref[...])

    bcf = dot(adjf, g25_ref[...])
    bcm = dot(adjm, g25_ref[...])

    bef = bef_ref[...]
    bem = bem_ref[...]
    aef = aef_ref[0, 0]
    aem = aem_ref[0, 0]
    hf = []
    hm = []
    for s in range(S):
        c0 = S * s
        accf = bcf[:, c0 * H:(c0 + 1) * H] * fts_f[:E]
        accm = bcm[:, c0 * H:(c0 + 1) * H] * fts_m[:E]
        for t in range(1, S):
            c = c0 + t
            accf += bcf[:, c * H:(c + 1) * H] * fts_f[t * E:(t + 1) * E]
            accm += bcm[:, c * H:(c + 1) * H] * fts_m[t * E:(t + 1) * E]
        accf += bef
        accm += bem
        hf.append(jnp.where(accf >= 0, accf, aef * accf))
        hm.append(jnp.where(accm >= 0, accm, aem * accm))

    hm4 = hm[S - 1]
    dstack = jnp.concatenate([hm[s] - hm4 for s in range(S - 1)], axis=0)
    z = dot(dstack, wt_ref[...]) + bt_ref[0, 0]
    sc = jax.nn.sigmoid(z)
    scb = dot(sc, ones_r_ref[...])
    readout = scb[:E] * hf[0]
    for s in range(1, S - 1):
        readout += scb[s * E:(s + 1) * E] * hf[s]

    u = dot(hf[S - 1][P:, :], wd_ref[...])
    rstack = jnp.concatenate(
        [readout[P:P + T], readout[P - 1:P - 1 + T], readout[P - 2:P - 2 + T]],
        axis=0)
    ustack = jnp.concatenate([u, u, u], axis=0)
    psum = dot(ustack * rstack, ones_c_ref[...]) + bd_ref[0, 0]
    sc0_ref[...] = psum[:T]
    sc1_ref[...] = psum[T:2 * T]
    sc2_ref[...] = psum[2 * T:]

    nmf = jnp.concatenate(
        [hm[0][P:], hm[1][P:], hm[2][P:],
         hf[0][P:], hf[1][P:], hf[2][P:]], axis=1)
    x = dot(nmf, w1_ref[...]) + b1_ref[...]
    x = jnp.where(x >= 0, x, a1_ref[...] * x)
    x = dot(x, w2_ref[...]) + b2_ref[...]
    x = jnp.where(x >= 0, x, a2_ref[...] * x)
    x = dot(x, w3_ref[...]) + b3_ref[...]
    x = jnp.where(x >= 0, x, a3_ref[...] * x)
    frec_ref[...] = x[:, :FEAT]
    mrec_ref[...] = x[:, FEAT:]


def kernel(features, motifs, adj_feat, adj_motif, W_em, b_em, a_em, W_ef,
           b_ef, a_ef, Wm1, bm1, am1, Wm2, bm2, am2, Wm3, bm3, am3, Wf1, bf1,
           af1, Wf2, bf2, af2, Wf3, bf3, af3, Wt, bt, Wd, bd):
    f32 = jnp.float32
    feat2 = features.reshape(B, S * FEAT)
    mot2 = motifs.reshape(B, S * MOT)
    adjf2 = adj_feat.reshape(B, S * S)
    adjm2 = adj_motif.reshape(B, S * S)

    def row(v, n):
        return v.reshape(1, n).astype(f32)

    g25 = jnp.kron(jnp.eye(S * S, dtype=f32), jnp.ones((1, H), f32))
    ones_r = jnp.ones((1, H), f32)
    ones_c = jnp.ones((H, 1), f32)

    zz = jnp.zeros((3 * H, H), f32)
    w1b = jnp.concatenate(
        [jnp.concatenate([Wm1.T, zz], axis=0),
         jnp.concatenate([zz, Wf1.T], axis=0)], axis=1)
    b1b = jnp.concatenate([row(bm1, H), row(bf1, H)], axis=1)
    a1b = jnp.concatenate(
        [jnp.broadcast_to(row(am1, 1), (1, H)),
         jnp.broadcast_to(row(af1, 1), (1, H))], axis=1)
    z2 = jnp.zeros((H, H), f32)
    w2b = jnp.concatenate(
        [jnp.concatenate([Wm2.T, z2], axis=0),
         jnp.concatenate([z2, Wf2.T], axis=0)], axis=1)
    b2b = jnp.concatenate([row(bm2, H), row(bf2, H)], axis=1)
    a2b = jnp.concatenate(
        [jnp.broadcast_to(row(am2, 1), (1, H)),
         jnp.broadcast_to(row(af2, 1), (1, H))], axis=1)
    w3b = jnp.concatenate(
        [jnp.concatenate([jnp.zeros((H, FEAT), f32), Wm3.T], axis=1),
         jnp.concatenate([Wf3.T, jnp.zeros((H, MOT), f32)], axis=1)],
        axis=0)
    b3b = jnp.concatenate([row(bf3, FEAT), row(bm3, MOT)], axis=1)
    a3b = jnp.concatenate(
        [jnp.broadcast_to(row(af3, 1), (1, FEAT)),
         jnp.broadcast_to(row(am3, 1), (1, MOT))], axis=1)

    w_args = (
        g25, ones_r, ones_c,
        W_ef.T, W_em.T, row(b_ef, H), row(b_em, H), row(a_ef, 1),
        row(a_em, 1),
        Wt.T, row(bt, 1), Wd[0], row(bd, 1),
        w1b, b1b, a1b, w2b, b2b, a2b, w3b, b3b, a3b,
    )

    grid = (B // T,)
    nb_prev = B // P

    def main2(t):
        return (t, 0)

    def prev2(t):
        return ((t * (T // P) - 1) % nb_prev, 0)

    def const(shape):
        return pl.BlockSpec(shape, lambda t: (0,) * len(shape))

    in_specs = [
        pl.BlockSpec((T, S * FEAT), main2),
        pl.BlockSpec((P, S * FEAT), prev2),
        pl.BlockSpec((T, S * MOT), main2),
        pl.BlockSpec((P, S * MOT), prev2),
        pl.BlockSpec((T, S * S), main2),
        pl.BlockSpec((P, S * S), prev2),
        pl.BlockSpec((T, S * S), main2),
        pl.BlockSpec((P, S * S), prev2),
    ] + [const(w.shape) for w in w_args]

    out_specs = [
        pl.BlockSpec((T, MOT), main2),
        pl.BlockSpec((T, FEAT), main2),
        pl.BlockSpec((T, 1), main2),
        pl.BlockSpec((T, 1), main2),
        pl.BlockSpec((T, 1), main2),
    ]
    out_shape = [
        jax.ShapeDtypeStruct((B, MOT), f32),
        jax.ShapeDtypeStruct((B, FEAT), f32),
        jax.ShapeDtypeStruct((B, 1), f32),
        jax.ShapeDtypeStruct((B, 1), f32),
        jax.ShapeDtypeStruct((B, 1), f32),
    ]

    mrec, frec, sc0, sc1, sc2 = pl.pallas_call(
        _fused_kernel,
        grid=grid,
        in_specs=in_specs,
        out_specs=out_specs,
        out_shape=out_shape,
    )(feat2, feat2, mot2, mot2, adjf2, adjf2, adjm2, adjm2, *w_args)

    logits = jnp.concatenate([sc0, sc1, sc2], axis=0)
    return (logits, mrec, frec)
